# Initial kernel scaffold; baseline (speedup 1.0000x reference)
import functools

import jax
import jax.numpy as jnp
from jax import lax
from jax.experimental import pallas as pl
from jax.experimental.pallas import tpu as pltpu
from jax.experimental.pallas import tpu_sc as plsc

N = 10000
E = 320000
D = 128
H = 128
OUT_DIM = 1
G = 16

NC = 2
NS = 16
NW = NC * NS
EPW = E // NW
CHUNK = 80
NCHUNK = EPW // CHUNK
RPT = N // NS
RBLK = 125
NRB = RPT // RBLK
DEGW = 16

_sc_mesh = plsc.VectorSubcoreMesh(core_axis_name="c", subcore_axis_name="s")


@functools.partial(
    pl.kernel,
    out_type=jax.ShapeDtypeStruct((NC, N, DEGW), jnp.float32),
    mesh=_sc_mesh,
    scratch_types=[
        pltpu.VMEM((NCHUNK, CHUNK), jnp.int32),
        pltpu.VMEM((CHUNK, DEGW), jnp.float32),
        pltpu.VMEM((RPT, DEGW), jnp.float32),
        pltpu.VMEM_SHARED((N, DEGW), jnp.float32),
    ],
)
def _deg_kernel(dst_hbm, ones_hbm, zeros_hbm, out_hbm, dst_v, ones_v, zbuf, acc):
    c = lax.axis_index("c")
    s = lax.axis_index("s")
    pltpu.sync_copy(zeros_hbm, zbuf)
    pltpu.sync_copy(zbuf, acc.at[pl.ds(s * RPT, RPT)])
    pltpu.sync_copy(dst_hbm.at[c, s], dst_v)
    pltpu.sync_copy(ones_hbm, ones_v)
    plsc.subcore_barrier()

    def body(i, carry):
        pltpu.sync_copy(ones_v, acc.at[dst_v.at[i]], add=True)
        return carry

    lax.fori_loop(0, NCHUNK, body, 0)
    plsc.subcore_barrier()
    pltpu.sync_copy(acc.at[pl.ds(s * RPT, RPT)], zbuf)
    pltpu.sync_copy(zbuf, out_hbm.at[c, pl.ds(s * RPT, RPT)])


@functools.partial(
    pl.kernel,
    out_type=jax.ShapeDtypeStruct((NC, N, H), jnp.float32),
    mesh=_sc_mesh,
    scratch_types=[
        pltpu.VMEM((NCHUNK, CHUNK), jnp.int32),
        pltpu.VMEM((NCHUNK, CHUNK), jnp.int32),
        pltpu.VMEM((CHUNK, H), jnp.float32),
        pltpu.VMEM((RBLK, H), jnp.float32),
        pltpu.VMEM_SHARED((N, H), jnp.float32),
        pltpu.SemaphoreType.DMA,
    ],
)
def _edge_scatter(u_hbm, src_hbm, dst_hbm, zeros_hbm, out_hbm,
                  src_v, dst_v, rows_v, zbuf, acc, sem):
    c = lax.axis_index("c")
    s = lax.axis_index("s")
    pltpu.sync_copy(zeros_hbm, zbuf)
    for r in range(NRB):
        pltpu.sync_copy(zbuf, acc.at[pl.ds(s * RPT + r * RBLK, RBLK)])
    pltpu.sync_copy(src_hbm.at[c, s], src_v)
    pltpu.sync_copy(dst_hbm.at[c, s], dst_v)
    plsc.subcore_barrier()

    def body(i, carry):
        pltpu.async_copy(u_hbm.at[src_v.at[i]], rows_v, sem).wait()
        pltpu.sync_copy(rows_v, acc.at[dst_v.at[i]], add=True)
        return carry

    lax.fori_loop(0, NCHUNK, body, 0)
    plsc.subcore_barrier()
    for r in range(NRB):
        sl = pl.ds(s * RPT + r * RBLK, RBLK)
        pltpu.sync_copy(acc.at[sl], zbuf)
        pltpu.sync_copy(zbuf, out_hbm.at[c, sl])


def _lrelu(v):
    return jnp.where(v > 0, v, 0.01 * v)


def _tc_pre_body(degp_ref, x_ref, wemb_ref, bemb_ref, wc_ref, u0_ref, dis_ref):
    deg = degp_ref[0][:, 0:1] + degp_ref[1][:, 0:1] + 1.0
    dis = lax.rsqrt(jnp.maximum(deg, 1e-12))
    h = jnp.dot(x_ref[...], wemb_ref[...], preferred_element_type=jnp.float32)
    h = h + bemb_ref[...]
    u0_ref[...] = dis * jnp.dot(h, wc_ref[...], preferred_element_type=jnp.float32)
    dis_ref[...] = dis


def _tc_mid_body(accp_ref, u_ref, dis_ref, bc_ref, g_ref, be_ref, wc_ref, un_ref):
    dis = dis_ref[...]
    z = dis * (accp_ref[0] + accp_ref[1] + u_ref[...]) + bc_ref[...]
    m = jnp.mean(z, axis=0, keepdims=True)
    zc = z - m
    v = jnp.mean(zc * zc, axis=0, keepdims=True)
    hh = _lrelu(zc * lax.rsqrt(v + 1e-5) * g_ref[...] + be_ref[...])
    un_ref[...] = dis * jnp.dot(hh, wc_ref[...], preferred_element_type=jnp.float32)


def _tc_post_body(accp_ref, u_ref, dis_ref, bc_ref, g_ref, be_ref, batch_ref,
                  wr1_ref, br1_ref, wr2_ref, br2_ref, out_ref):
    z = dis_ref[...] * (accp_ref[0] + accp_ref[1] + u_ref[...]) + bc_ref[...]
    m = jnp.mean(z, axis=0, keepdims=True)
    zc = z - m
    v = jnp.mean(zc * zc, axis=0, keepdims=True)
    hh = _lrelu(zc * lax.rsqrt(v + 1e-5) * g_ref[...] + be_ref[...])
    gids = lax.broadcasted_iota(jnp.int32, (G, N), 0)
    oh = (gids == batch_ref[...]).astype(jnp.float32)
    pooled = jnp.dot(oh, hh, preferred_element_type=jnp.float32)
    r = _lrelu(jnp.dot(pooled, wr1_ref[...], preferred_element_type=jnp.float32)
               + br1_ref[...])
    out_ref[...] = (jnp.dot(r, wr2_ref[...], preferred_element_type=jnp.float32)
                    + br2_ref[...])


_tc_pre = pl.pallas_call(
    _tc_pre_body,
    out_shape=[jax.ShapeDtypeStruct((N, H), jnp.float32),
               jax.ShapeDtypeStruct((N, 1), jnp.float32)],
)

_tc_mid = pl.pallas_call(
    _tc_mid_body,
    out_shape=jax.ShapeDtypeStruct((N, H), jnp.float32),
)

_tc_post = pl.pallas_call(
    _tc_post_body,
    out_shape=jax.ShapeDtypeStruct((G, OUT_DIM), jnp.float32),
)


def kernel(x, edge_index, batch, W_emb, b_emb, Wc0, bc0, g0, be0, Wc1, bc1,
           g1, be1, Wc2, bc2, g2, be2, Wr1, br1, Wr2, br2):
    src = edge_index[0].reshape(NC, NS, NCHUNK, CHUNK)
    dst = edge_index[1].reshape(NC, NS, NCHUNK, CHUNK)
    zeros_rows = jnp.zeros((RBLK, H), jnp.float32)
    zeros_deg = jnp.zeros((RPT, DEGW), jnp.float32)
    ones_deg = jnp.ones((CHUNK, DEGW), jnp.float32)

    degp = _deg_kernel(dst, ones_deg, zeros_deg)
    u0, dis = _tc_pre(degp, x, W_emb, b_emb.reshape(1, H), Wc0)

    acc = _edge_scatter(u0, src, dst, zeros_rows)
    u1 = _tc_mid(acc, u0, dis, bc0.reshape(1, H), g0.reshape(1, H),
                 be0.reshape(1, H), Wc1)
    acc = _edge_scatter(u1, src, dst, zeros_rows)
    u2 = _tc_mid(acc, u1, dis, bc1.reshape(1, H), g1.reshape(1, H),
                 be1.reshape(1, H), Wc2)
    acc = _edge_scatter(u2, src, dst, zeros_rows)
    out = _tc_post(acc, u2, dis, bc2.reshape(1, H), g2.reshape(1, H),
                   be2.reshape(1, H), batch.reshape(1, N),
                   Wr1, br1.reshape(1, H // 2), Wr2, br2.reshape(1, OUT_DIM))
    return out


# speedup vs baseline: 23.1458x; 23.1458x over previous
import functools

import jax
import jax.numpy as jnp
from jax import lax
from jax.experimental import pallas as pl
from jax.experimental.pallas import tpu as pltpu
from jax.experimental.pallas import tpu_sc as plsc

N = 10000
E = 320000
D = 128
H = 128
OUT_DIM = 1
G = 16

NC = 2
NS = 16
EPT = E // NS
CHUNK = 125
NCHUNK = EPT // CHUNK
NBUF = 8
NPAD = 10112
RPT = NPAD // NS
NP = 4
WP = H // NP
DEGW = 16


def _deg_body(dst_hbm, ones_hbm, zeros_hbm, out_hbm, dst_v, ones_v, zbuf, acc,
              sem):
    c = lax.axis_index("c")
    s = lax.axis_index("s")
    own = pl.ds(s * RPT, RPT)
    pltpu.sync_copy(zeros_hbm, zbuf)
    pltpu.sync_copy(zbuf, acc.at[own])
    pltpu.sync_copy(dst_hbm.at[s], dst_v)
    pltpu.sync_copy(ones_hbm, ones_v)
    plsc.subcore_barrier()

    def body(j, carry):
        base = c * (NCHUNK // 2) + j * NBUF
        for k in range(NBUF):
            pltpu.make_async_copy(
                ones_v, acc.at[dst_v.at[base + k]], sem).start(add=True)
        for k in range(NBUF):
            pltpu.make_async_copy(
                ones_v, acc.at[dst_v.at[base + k]], sem).wait()
        return carry

    lax.fori_loop(0, NCHUNK // 2 // NBUF, body, 0)
    plsc.subcore_barrier()
    pltpu.sync_copy(acc.at[own], zbuf)
    pltpu.sync_copy(zbuf, out_hbm.at[c, own])


def _edge_body(u4_hbm, src4p_hbm, dst_hbm, zeros_hbm,
               out_hbm, src_v, dst_v, r0, r1, r2, r3, r4, r5, r6, r7, zwb,
               acc, g0, g1, g2, g3, g4, g5, g6, g7,
               s0, s1, s2, s3, s4, s5, s6, s7):
    c = lax.axis_index("c")
    s = lax.axis_index("s")
    own = pl.ds(s * RPT, RPT)
    rows = (r0, r1, r2, r3, r4, r5, r6, r7)
    gsem = (g0, g1, g2, g3, g4, g5, g6, g7)
    ssem = (s0, s1, s2, s3, s4, s5, s6, s7)
    pltpu.sync_copy(dst_hbm.at[s], dst_v)

    def do_pass(p):
        u_hbm = u4_hbm
        pltpu.sync_copy(src4p_hbm.at[p, s], src_v)
        pltpu.sync_copy(zeros_hbm, zwb)
        pltpu.sync_copy(zwb, acc.at[own])
        plsc.subcore_barrier()

        def body(j, carry):
            base = j * NBUF
            for k in range(NBUF):
                @pl.when(j > 0)
                def _():
                    pltpu.make_async_copy(
                        rows[k], acc.at[dst_v.at[0]], ssem[k]).wait()

                pltpu.make_async_copy(
                    u_hbm.at[src_v.at[base + k]], rows[k], gsem[k]).start()
            for k in range(NBUF):
                pltpu.make_async_copy(
                    u_hbm.at[src_v.at[base + k]], rows[k], gsem[k]).wait()
                pltpu.make_async_copy(
                    rows[k], acc.at[dst_v.at[base + k]], ssem[k]).start(add=True)
            return carry

        lax.fori_loop(0, NCHUNK // NBUF, body, 0)
        for k in range(NBUF):
            pltpu.make_async_copy(rows[k], acc.at[dst_v.at[0]], ssem[k]).wait()
        plsc.subcore_barrier()
        pltpu.sync_copy(acc.at[own], zwb)
        pltpu.sync_copy(zwb, out_hbm.at[p, own])

    @pl.when(c == 0)
    def _():
        do_pass(0)
        do_pass(1)

    @pl.when(c == 1)
    def _():
        do_pass(2)
        do_pass(3)


@functools.lru_cache(maxsize=None)
def _sc_kernels():
    mesh = plsc.VectorSubcoreMesh(core_axis_name="c", subcore_axis_name="s")
    params = pltpu.CompilerParams(use_tc_tiling_on_sc=False)
    deg_kernel = pl.kernel(
        _deg_body,
        compiler_params=params,
        out_type=jax.ShapeDtypeStruct((NC, NPAD, DEGW), jnp.float32),
        mesh=mesh,
        scratch_types=[
            pltpu.VMEM((NCHUNK, CHUNK), jnp.int32),
            pltpu.VMEM((CHUNK, DEGW), jnp.float32),
            pltpu.VMEM((RPT, DEGW), jnp.float32),
            pltpu.VMEM_SHARED((NPAD, DEGW), jnp.float32),
            pltpu.SemaphoreType.DMA,
        ],
    )
    edge_kernel = pl.kernel(
        _edge_body,
        compiler_params=params,
        out_type=jax.ShapeDtypeStruct((NP, NPAD, WP), jnp.float32),
        mesh=mesh,
        scratch_types=(
            [pltpu.VMEM((NCHUNK, CHUNK), jnp.int32)] * 2
            + [pltpu.VMEM((CHUNK, WP), jnp.float32)] * NBUF
            + [pltpu.VMEM((RPT, WP), jnp.float32)]
            + [pltpu.VMEM_SHARED((NPAD, WP), jnp.float32)]
            + [pltpu.SemaphoreType.DMA] * (2 * NBUF)
        ),
    )
    return deg_kernel, edge_kernel


def _lrelu(v):
    return jnp.where(v > 0, v, 0.01 * v)


def _tc_pre_body(degp_ref, x_ref, wemb_ref, bemb_ref, wc_ref, u0_ref, dis_ref):
    deg = degp_ref[0][:N, 0:1] + degp_ref[1][:N, 0:1] + 1.0
    dis = lax.rsqrt(jnp.maximum(deg, 1e-12))
    h = jnp.dot(x_ref[...], wemb_ref[...], preferred_element_type=jnp.float32)
    h = h + bemb_ref[...]
    u0_ref[...] = dis * jnp.dot(h, wc_ref[...], preferred_element_type=jnp.float32)
    dis_ref[...] = dis


def _cat_acc(accp_ref):
    return jnp.concatenate([accp_ref[p][:N] for p in range(NP)], axis=1)


def _tc_mid_body(accp_ref, u_ref, dis_ref, bc_ref, g_ref, be_ref, wc_ref, un_ref):
    dis = dis_ref[...]
    z = dis * (_cat_acc(accp_ref) + u_ref[...]) + bc_ref[...]
    m = jnp.mean(z, axis=0, keepdims=True)
    zc = z - m
    v = jnp.mean(zc * zc, axis=0, keepdims=True)
    hh = _lrelu(zc * lax.rsqrt(v + 1e-5) * g_ref[...] + be_ref[...])
    un_ref[...] = dis * jnp.dot(hh, wc_ref[...], preferred_element_type=jnp.float32)


def _tc_post_body(accp_ref, u_ref, dis_ref, bc_ref, g_ref, be_ref, batch_ref,
                  wr1_ref, br1_ref, wr2_ref, br2_ref, out_ref):
    z = dis_ref[...] * (_cat_acc(accp_ref) + u_ref[...]) + bc_ref[...]
    m = jnp.mean(z, axis=0, keepdims=True)
    zc = z - m
    v = jnp.mean(zc * zc, axis=0, keepdims=True)
    hh = _lrelu(zc * lax.rsqrt(v + 1e-5) * g_ref[...] + be_ref[...])
    seg = batch_ref[...]
    pooled = jnp.concatenate(
        [jnp.sum(jnp.where(seg == g, hh, 0.0), axis=0, keepdims=True)
         for g in range(G)], axis=0)
    r = _lrelu(jnp.dot(pooled, wr1_ref[...], preferred_element_type=jnp.float32)
               + br1_ref[...])
    out_ref[...] = (jnp.dot(r, wr2_ref[...], preferred_element_type=jnp.float32)
                    + br2_ref[...])


_tc_pre = pl.pallas_call(
    _tc_pre_body,
    out_shape=[jax.ShapeDtypeStruct((N, H), jnp.float32),
               jax.ShapeDtypeStruct((N, 1), jnp.float32)],
)

_tc_mid = pl.pallas_call(
    _tc_mid_body,
    out_shape=jax.ShapeDtypeStruct((N, H), jnp.float32),
)

_tc_post = pl.pallas_call(
    _tc_post_body,
    out_shape=jax.ShapeDtypeStruct((G, OUT_DIM), jnp.float32),
)


def kernel(x, edge_index, batch, W_emb, b_emb, Wc0, bc0, g0, be0, Wc1, bc1,
           g1, be1, Wc2, bc2, g2, be2, Wr1, br1, Wr2, br2):
    src4 = (edge_index[0] * NP).reshape(1, NS, NCHUNK, CHUNK)
    src4p = src4 + jnp.arange(NP, dtype=jnp.int32).reshape(NP, 1, 1, 1)
    dst = edge_index[1].reshape(NS, NCHUNK, CHUNK)
    zeros_rows = jnp.zeros((RPT, WP), jnp.float32)
    zeros_deg = jnp.zeros((RPT, DEGW), jnp.float32)
    ones_deg = jnp.ones((CHUNK, DEGW), jnp.float32)

    deg_kernel, edge_scatter = _sc_kernels()
    degp = deg_kernel(dst, ones_deg, zeros_deg)
    u, dis = _tc_pre(degp, x, W_emb, b_emb.reshape(1, H), Wc0)

    acc = edge_scatter(u.reshape(N * NP, WP), src4p, dst, zeros_rows)
    u = _tc_mid(acc, u, dis, bc0.reshape(1, H), g0.reshape(1, H),
                be0.reshape(1, H), Wc1)
    acc = edge_scatter(u.reshape(N * NP, WP), src4p, dst, zeros_rows)
    u = _tc_mid(acc, u, dis, bc1.reshape(1, H), g1.reshape(1, H),
                be1.reshape(1, H), Wc2)
    acc = edge_scatter(u.reshape(N * NP, WP), src4p, dst, zeros_rows)
    out = _tc_post(acc, u, dis, bc2.reshape(1, H), g2.reshape(1, H),
                   be2.reshape(1, H), batch.reshape(N, 1),
                   Wr1, br1.reshape(1, H // 2), Wr2, br2.reshape(1, OUT_DIM))
    return out
